# baseline (speedup 1.0000x reference)
import functools

import jax
import jax.numpy as jnp
from jax import lax
from jax.experimental import pallas as pl
from jax.experimental.pallas import tpu as pltpu
from jax.experimental.pallas import tpu_sc as plsc

N = 10000
E = 160000
DIN = 128
DH = 128
G4 = 4 * DH
B = 2
T = 4
GALL = B * T

NC = 2
NS = 16
NW = NC * NS

K = 128
NBUF = 2
EP = 163840
NCH = EP // K
IT0 = 80
IT1 = 0
HIT = 40
ITD = NCH // NW
RPT = 632
NP = NS * RPT
ACC_ROWS = NP

_MESH = dict(core_axis_name="c", subcore_axis_name="s", num_cores=NC,
             num_subcores=NS)


def _deg_kernel():
    @functools.partial(
        pl.kernel,
        out_type=jax.ShapeDtypeStruct((NC, NP, 16), jnp.float32),
        mesh=plsc.VectorSubcoreMesh(**_MESH),
        scratch_types=[
            pltpu.VMEM((ITD, K), jnp.int32),
            pltpu.VMEM((K, 16), jnp.float32),
            pltpu.VMEM_SHARED((ACC_ROWS, 16), jnp.float32),
        ],
    )
    def deg(dstp3, z16, out, dst_buf, ones_v, acc):
        c = lax.axis_index("c")
        s = lax.axis_index("s")
        w = c * NS + s
        rbase = s * RPT
        for j in range(K):
            ones_v[j, :] = jnp.ones((16,), jnp.float32)
        pltpu.sync_copy(dstp3.at[pl.ds(w * ITD, ITD)], dst_buf)
        pltpu.sync_copy(z16.at[pl.ds(rbase, RPT)], acc.at[pl.ds(rbase, RPT)])
        plsc.subcore_barrier()

        @pl.loop(0, ITD)
        def _(i):
            pltpu.sync_copy(ones_v, acc.at[dst_buf.at[i]], add=True)

        plsc.subcore_barrier()
        pltpu.sync_copy(acc.at[pl.ds(rbase, RPT)],
                        out.at[c, pl.ds(rbase, RPT)])

    return deg


def _edge_scatter_kernel(num_g):
    @functools.partial(
        pl.kernel,
        out_type=jax.ShapeDtypeStruct((NC, num_g, NP, DH), jnp.float32),
        mesh=plsc.VectorSubcoreMesh(**_MESH),
        scratch_types=[
            pltpu.VMEM((HIT, K), jnp.int32),
            pltpu.VMEM((HIT, K), jnp.int32),
        ] + [pltpu.VMEM((K, DH), jnp.float32)] * NBUF
          + [pltpu.VMEM_SHARED((ACC_ROWS, DH), jnp.float32)]
          + [pltpu.SemaphoreType.DMA] * NBUF,
    )
    def scat(tables, srcg3, dstp3, zrows, out, src_buf, dst_buf,
             *rest):
        rows = rest[:NBUF]
        acc = rest[NBUF]
        sems = rest[NBUF + 1:]
        c = lax.axis_index("c")
        s = lax.axis_index("s")
        rbase = s * RPT

        def issue(m, par):
            pltpu.async_copy(tables.at[src_buf.at[m]], rows[par], sems[par])

        def wait_scat(j, par):
            pltpu.make_async_copy(
                tables.at[src_buf.at[j]], rows[par], sems[par]).wait()
            pltpu.sync_copy(rows[par], acc.at[dst_buf.at[j]], add=True)

        def pipeline(iters):
            for m in range(NBUF - 1):
                issue(m, m)

            @pl.loop(0, iters - NBUF, step=NBUF)
            def _(jb):
                for par in range(NBUF):
                    j = jb + par
                    issue(j + NBUF - 1, (par + NBUF - 1) % NBUF)
                    wait_scat(j, par)

            issue(iters - 1, (iters - 1) % NBUF)
            for m in range(iters - NBUF, iters):
                wait_scat(m, m % NBUF)

        for g in range(num_g):
            pltpu.sync_copy(zrows.at[pl.ds(rbase, RPT)],
                            acc.at[pl.ds(rbase, RPT)])
            plsc.subcore_barrier()

            @pl.when(c == 0)
            def _():
                for h in range(IT0 // HIT):
                    cb = s * IT0 + h * HIT
                    pltpu.sync_copy(dstp3.at[pl.ds(cb, HIT)], dst_buf)
                    pltpu.sync_copy(srcg3.at[g, pl.ds(cb, HIT)], src_buf)
                    pipeline(HIT)

            if IT1:
                @pl.when(c == 1)
                def _():
                    cb = NS * IT0 + s * IT1
                    pltpu.sync_copy(dstp3.at[pl.ds(cb, IT1)],
                                    dst_buf.at[pl.ds(0, IT1)])
                    pltpu.sync_copy(srcg3.at[g, pl.ds(cb, IT1)],
                                    src_buf.at[pl.ds(0, IT1)])
                    pipeline(IT1)

            plsc.subcore_barrier()
            pltpu.sync_copy(acc.at[pl.ds(rbase, RPT)],
                            out.at[c, g, pl.ds(rbase, RPT)])
            plsc.subcore_barrier()

    return scat


R = 2000
NBLK = N // R


def _scale_body(degp, x, xs, dinv_b, sn_b):
    deg = degp[0, :, 0:1] + degp[1, :, 0:1] + 2.0
    dinv = lax.rsqrt(deg)
    db = jnp.broadcast_to(dinv, (R, DIN))
    xs[0] = dinv * x[0]
    dinv_b[...] = db
    sn_b[...] = 2.0 * db * db


def _scale_call(degp, xf):
    return pl.pallas_call(
        _scale_body,
        grid=(GALL, NBLK),
        in_specs=[
            pl.BlockSpec((NC, R, 16), lambda g, n: (0, n, 0)),
            pl.BlockSpec((1, R, DIN), lambda g, n: (g, n, 0)),
        ],
        out_specs=[
            pl.BlockSpec((1, R, DIN), lambda g, n: (g, n, 0)),
            pl.BlockSpec((R, DIN), lambda g, n: (n, 0)),
            pl.BlockSpec((R, DIN), lambda g, n: (n, 0)),
        ],
        out_shape=[
            jax.ShapeDtypeStruct((GALL, N, DIN), jnp.float32),
            jax.ShapeDtypeStruct((N, DIN), jnp.float32),
            jax.ShapeDtypeStruct((N, DIN), jnp.float32),
        ],
    )(degp, xf)


def _u_body(parts, x, dinv_b, sn_b, wx, bias, u):
    p = parts[0, 0] + parts[1, 0]
    y = dinv_b[...] * p + sn_b[...] * x[0]
    u[0] = jnp.dot(y, wx[...], preferred_element_type=jnp.float32) + bias[...]


def _u_call(parts8, xf, dinv_b, sn_b, wx, bias2):
    return pl.pallas_call(
        _u_body,
        grid=(GALL, NBLK),
        in_specs=[
            pl.BlockSpec((NC, 1, R, DIN), lambda g, n: (0, g, n, 0)),
            pl.BlockSpec((1, R, DIN), lambda g, n: (g, n, 0)),
            pl.BlockSpec((R, DIN), lambda g, n: (n, 0)),
            pl.BlockSpec((R, DIN), lambda g, n: (n, 0)),
            pl.BlockSpec((DIN, G4), lambda g, n: (0, 0)),
            pl.BlockSpec((1, G4), lambda g, n: (0, 0)),
        ],
        out_specs=pl.BlockSpec((1, R, G4), lambda g, n: (g, n, 0)),
        out_shape=jax.ShapeDtypeStruct((GALL, N, G4), jnp.float32),
    )(parts8, xf, dinv_b, sn_b, wx, bias2)


def _gates(conv, c_prev):
    i_ = jax.nn.sigmoid(conv[:, 0:DH])
    f_ = jax.nn.sigmoid(conv[:, DH:2 * DH])
    o_ = jax.nn.sigmoid(conv[:, 2 * DH:3 * DH])
    g_ = jnp.tanh(conv[:, 3 * DH:4 * DH])
    c_new = f_ * c_prev + i_ * g_
    h_new = o_ * jnp.tanh(c_new)
    return h_new, c_new


def _step0_body(u, dinv_b, h, c, hs):
    conv = u[0]
    h_new, c_new = _gates(conv, jnp.zeros((R, DH), jnp.float32))
    h[0] = h_new
    c[0] = c_new
    hs[0] = dinv_b[...] * h_new


def _step0_call(u_all):
    return pl.pallas_call(
        _step0_body,
        grid=(B, NBLK),
        in_specs=[
            pl.BlockSpec((1, R, G4), lambda b, n: (b * T, n, 0)),
            pl.BlockSpec((R, DH), lambda b, n: (n, 0)),
        ],
        out_specs=[
            pl.BlockSpec((1, R, DH), lambda b, n: (b, n, 0)),
            pl.BlockSpec((1, R, DH), lambda b, n: (b, n, 0)),
            pl.BlockSpec((1, R, DH), lambda b, n: (b, n, 0)),
        ],
        out_shape=[
            jax.ShapeDtypeStruct((B, N, DH), jnp.float32),
            jax.ShapeDtypeStruct((B, N, DH), jnp.float32),
            jax.ShapeDtypeStruct((B, N, DH), jnp.float32),
        ],
    )


def _step_body(u, parts, h_in, c_in, dinv_b, sn_b, wh, h, c, hs):
    p = parts[0, 0] + parts[1, 0]
    z = dinv_b[...] * p + sn_b[...] * h_in[0]
    conv = u[0] + jnp.dot(z, wh[...], preferred_element_type=jnp.float32)
    h_new, c_new = _gates(conv, c_in[0])
    h[0] = h_new
    c[0] = c_new
    hs[0] = dinv_b[...] * h_new


def _step_call(t):
    return pl.pallas_call(
        _step_body,
        grid=(B, NBLK),
        in_specs=[
            pl.BlockSpec((1, R, G4), lambda b, n: (b * T + t, n, 0)),
            pl.BlockSpec((NC, 1, R, DH), lambda b, n: (0, b, n, 0)),
            pl.BlockSpec((1, R, DH), lambda b, n: (b, n, 0)),
            pl.BlockSpec((1, R, DH), lambda b, n: (b, n, 0)),
            pl.BlockSpec((R, DH), lambda b, n: (n, 0)),
            pl.BlockSpec((R, DH), lambda b, n: (n, 0)),
            pl.BlockSpec((DH, G4), lambda b, n: (0, 0)),
        ],
        out_specs=[
            pl.BlockSpec((1, R, DH), lambda b, n: (b, n, 0)),
            pl.BlockSpec((1, R, DH), lambda b, n: (b, n, 0)),
            pl.BlockSpec((1, R, DH), lambda b, n: (b, n, 0)),
        ],
        out_shape=[
            jax.ShapeDtypeStruct((B, N, DH), jnp.float32),
            jax.ShapeDtypeStruct((B, N, DH), jnp.float32),
            jax.ShapeDtypeStruct((B, N, DH), jnp.float32),
        ],
    )


def kernel(input_tensor, edge_index, W, b):
    xf = input_tensor.reshape(GALL, N, DIN)
    src = edge_index[0]
    dst = edge_index[1]
    pad = EP - E
    src_p = jnp.concatenate([src, jnp.zeros((pad,), jnp.int32)])
    dst_p = jnp.concatenate([dst, jnp.full((pad,), N, jnp.int32)])
    goff = (jnp.arange(GALL, dtype=jnp.int32) * N)[:, None]
    srcg = (src_p[None, :] + goff).reshape(GALL, NCH, K)
    dstp3 = dst_p.reshape(NCH, K)

    z16 = jnp.zeros((NP, 16), jnp.float32)
    zrows = jnp.zeros((NP, DH), jnp.float32)
    wx = W[:DIN, :]
    wh = W[DIN:, :]
    bias2 = b.reshape(1, G4)

    degp = _deg_kernel()(dstp3, z16)
    xs, dinv_b, sn_b = _scale_call(degp, xf)

    parts8 = _edge_scatter_kernel(GALL)(
        xs.reshape(GALL * N, DIN), srcg, dstp3, zrows)
    u_all = _u_call(parts8, xf, dinv_b, sn_b, wx, bias2)

    h, c, hs = _step0_call(u_all)(u_all, dinv_b)
    outs = [h]
    scat2 = _edge_scatter_kernel(B)
    for t in range(1, T):
        parts2 = scat2(hs.reshape(B * N, DH), srcg, dstp3, zrows)
        h, c, hs = _step_call(t)(u_all, parts2, h, c, dinv_b, sn_b, wh)
        outs.append(h)
    return jnp.stack(outs, axis=1)


# speedup vs baseline: 3.1518x; 3.1518x over previous
import functools

import jax
import jax.numpy as jnp
from jax import lax
from jax.experimental import pallas as pl
from jax.experimental.pallas import tpu as pltpu
from jax.experimental.pallas import tpu_sc as plsc

N = 10000
E = 160000
DIN = 128
DH = 128
G4 = 4 * DH
B = 2
T = 4
GALL = B * T

NC = 2
NS = 16
NW = NC * NS

K = 128
NBUF = 2
EP = 163840
EPT = EP // NW
ITERS = EPT // K
RPT = 632
NP = NS * RPT
ACC_ROWS = NP

_MESH = dict(core_axis_name="c", subcore_axis_name="s", num_cores=NC,
             num_subcores=NS)


def _deg_kernel():
    @functools.partial(
        pl.kernel,
        out_type=jax.ShapeDtypeStruct((NC, NP, 16), jnp.float32),
        mesh=plsc.VectorSubcoreMesh(**_MESH),
        scratch_types=[
            pltpu.VMEM((ITERS, K), jnp.int32),
            pltpu.VMEM((K, 16), jnp.float32),
            pltpu.VMEM_SHARED((ACC_ROWS, 16), jnp.float32),
        ],
    )
    def deg(dstp3, z16, out, dst_buf, ones_v, acc):
        c = lax.axis_index("c")
        s = lax.axis_index("s")
        w = c * NS + s
        rbase = s * RPT
        for j in range(K):
            ones_v[j, :] = jnp.ones((16,), jnp.float32)
        pltpu.sync_copy(dstp3.at[w], dst_buf)
        pltpu.sync_copy(z16.at[pl.ds(rbase, RPT)], acc.at[pl.ds(rbase, RPT)])
        plsc.subcore_barrier()

        @pl.loop(0, ITERS)
        def _(i):
            pltpu.sync_copy(ones_v, acc.at[dst_buf.at[i]], add=True)

        plsc.subcore_barrier()
        pltpu.sync_copy(acc.at[pl.ds(rbase, RPT)],
                        out.at[c, pl.ds(rbase, RPT)])

    return deg


def _edge_scatter_kernel(num_g):
    @functools.partial(
        pl.kernel,
        out_type=jax.ShapeDtypeStruct((NC, num_g, NP, DH), jnp.float32),
        mesh=plsc.VectorSubcoreMesh(**_MESH),
        scratch_types=[
            pltpu.VMEM((ITERS, K), jnp.int32),
            pltpu.VMEM((ITERS, K), jnp.int32),
        ] + [pltpu.VMEM((K, DH), jnp.float32)] * NBUF
          + [pltpu.VMEM_SHARED((ACC_ROWS, DH), jnp.float32)]
          + [pltpu.SemaphoreType.DMA] * NBUF,
    )
    def scat(tables, srcg4, dstp3, zrows, out, src_buf, dst_buf,
             *rest):
        rows = rest[:NBUF]
        acc = rest[NBUF]
        sems = rest[NBUF + 1:]
        c = lax.axis_index("c")
        s = lax.axis_index("s")
        w = c * NS + s
        rbase = s * RPT

        def issue(m, par):
            pltpu.async_copy(tables.at[src_buf.at[m]], rows[par], sems[par])

        def wait_scat(j, par):
            pltpu.make_async_copy(
                tables.at[src_buf.at[j]], rows[par], sems[par]).wait()
            pltpu.sync_copy(rows[par], acc.at[dst_buf.at[j]], add=True)

        pltpu.sync_copy(dstp3.at[w], dst_buf)
        for g in range(num_g):
            pltpu.sync_copy(srcg4.at[g, w], src_buf)
            pltpu.sync_copy(zrows.at[pl.ds(rbase, RPT)],
                            acc.at[pl.ds(rbase, RPT)])
            plsc.subcore_barrier()

            for m in range(NBUF - 1):
                issue(m, m)

            @pl.loop(0, ITERS - NBUF, step=NBUF)
            def _(jb):
                for par in range(NBUF):
                    j = jb + par
                    issue(j + NBUF - 1, (par + NBUF - 1) % NBUF)
                    wait_scat(j, par)

            issue(ITERS - 1, (ITERS - 1) % NBUF)
            for m in range(ITERS - NBUF, ITERS):
                wait_scat(m, m % NBUF)

            plsc.subcore_barrier()
            pltpu.sync_copy(acc.at[pl.ds(rbase, RPT)],
                            out.at[c, g, pl.ds(rbase, RPT)])
            plsc.subcore_barrier()

    return scat


R = 2000
NBLK = N // R


def _scale_body(degp, x, xs, dinv_b, sn_b):
    deg = degp[0, :, 0:1] + degp[1, :, 0:1] + 2.0
    dinv = lax.rsqrt(deg)
    db = jnp.broadcast_to(dinv, (R, DIN))
    xs[0] = dinv * x[0]
    dinv_b[...] = db
    sn_b[...] = 2.0 * db * db


def _scale_call(degp, xf):
    return pl.pallas_call(
        _scale_body,
        grid=(GALL, NBLK),
        in_specs=[
            pl.BlockSpec((NC, R, 16), lambda g, n: (0, n, 0)),
            pl.BlockSpec((1, R, DIN), lambda g, n: (g, n, 0)),
        ],
        out_specs=[
            pl.BlockSpec((1, R, DIN), lambda g, n: (g, n, 0)),
            pl.BlockSpec((R, DIN), lambda g, n: (n, 0)),
            pl.BlockSpec((R, DIN), lambda g, n: (n, 0)),
        ],
        out_shape=[
            jax.ShapeDtypeStruct((GALL, N, DIN), jnp.float32),
            jax.ShapeDtypeStruct((N, DIN), jnp.float32),
            jax.ShapeDtypeStruct((N, DIN), jnp.float32),
        ],
    )(degp, xf)


def _u_body(parts, x, dinv_b, sn_b, wx, bias, u):
    p = parts[0, 0] + parts[1, 0]
    y = dinv_b[...] * p + sn_b[...] * x[0]
    u[0] = jnp.dot(y, wx[...], preferred_element_type=jnp.float32) + bias[...]


def _u_call(parts8, xf, dinv_b, sn_b, wx, bias2):
    return pl.pallas_call(
        _u_body,
        grid=(GALL, NBLK),
        in_specs=[
            pl.BlockSpec((NC, 1, R, DIN), lambda g, n: (0, g, n, 0)),
            pl.BlockSpec((1, R, DIN), lambda g, n: (g, n, 0)),
            pl.BlockSpec((R, DIN), lambda g, n: (n, 0)),
            pl.BlockSpec((R, DIN), lambda g, n: (n, 0)),
            pl.BlockSpec((DIN, G4), lambda g, n: (0, 0)),
            pl.BlockSpec((1, G4), lambda g, n: (0, 0)),
        ],
        out_specs=pl.BlockSpec((1, R, G4), lambda g, n: (g, n, 0)),
        out_shape=jax.ShapeDtypeStruct((GALL, N, G4), jnp.float32),
    )(parts8, xf, dinv_b, sn_b, wx, bias2)


def _gates(conv, c_prev):
    i_ = jax.nn.sigmoid(conv[:, 0:DH])
    f_ = jax.nn.sigmoid(conv[:, DH:2 * DH])
    o_ = jax.nn.sigmoid(conv[:, 2 * DH:3 * DH])
    g_ = jnp.tanh(conv[:, 3 * DH:4 * DH])
    c_new = f_ * c_prev + i_ * g_
    h_new = o_ * jnp.tanh(c_new)
    return h_new, c_new


def _step0_body(u, dinv_b, h, c, hs):
    conv = u[0]
    h_new, c_new = _gates(conv, jnp.zeros((R, DH), jnp.float32))
    h[0] = h_new
    c[0] = c_new
    hs[0] = dinv_b[...] * h_new


def _step0_call(u_all):
    return pl.pallas_call(
        _step0_body,
        grid=(B, NBLK),
        in_specs=[
            pl.BlockSpec((1, R, G4), lambda b, n: (b * T, n, 0)),
            pl.BlockSpec((R, DH), lambda b, n: (n, 0)),
        ],
        out_specs=[
            pl.BlockSpec((1, R, DH), lambda b, n: (b, n, 0)),
            pl.BlockSpec((1, R, DH), lambda b, n: (b, n, 0)),
            pl.BlockSpec((1, R, DH), lambda b, n: (b, n, 0)),
        ],
        out_shape=[
            jax.ShapeDtypeStruct((B, N, DH), jnp.float32),
            jax.ShapeDtypeStruct((B, N, DH), jnp.float32),
            jax.ShapeDtypeStruct((B, N, DH), jnp.float32),
        ],
    )


def _step_body(u, parts, h_in, c_in, dinv_b, sn_b, wh, h, c, hs):
    p = parts[0, 0] + parts[1, 0]
    z = dinv_b[...] * p + sn_b[...] * h_in[0]
    conv = u[0] + jnp.dot(z, wh[...], preferred_element_type=jnp.float32)
    h_new, c_new = _gates(conv, c_in[0])
    h[0] = h_new
    c[0] = c_new
    hs[0] = dinv_b[...] * h_new


def _step_call(t):
    return pl.pallas_call(
        _step_body,
        grid=(B, NBLK),
        in_specs=[
            pl.BlockSpec((1, R, G4), lambda b, n: (b * T + t, n, 0)),
            pl.BlockSpec((NC, 1, R, DH), lambda b, n: (0, b, n, 0)),
            pl.BlockSpec((1, R, DH), lambda b, n: (b, n, 0)),
            pl.BlockSpec((1, R, DH), lambda b, n: (b, n, 0)),
            pl.BlockSpec((R, DH), lambda b, n: (n, 0)),
            pl.BlockSpec((R, DH), lambda b, n: (n, 0)),
            pl.BlockSpec((DH, G4), lambda b, n: (0, 0)),
        ],
        out_specs=[
            pl.BlockSpec((1, R, DH), lambda b, n: (b, n, 0)),
            pl.BlockSpec((1, R, DH), lambda b, n: (b, n, 0)),
            pl.BlockSpec((1, R, DH), lambda b, n: (b, n, 0)),
        ],
        out_shape=[
            jax.ShapeDtypeStruct((B, N, DH), jnp.float32),
            jax.ShapeDtypeStruct((B, N, DH), jnp.float32),
            jax.ShapeDtypeStruct((B, N, DH), jnp.float32),
        ],
    )


def kernel(input_tensor, edge_index, W, b):
    xf = input_tensor.reshape(GALL, N, DIN)
    src = edge_index[0]
    dst = edge_index[1]
    pad = EP - E
    src_p = jnp.concatenate([src, jnp.arange(pad, dtype=jnp.int32)])
    dst_p = jnp.concatenate([dst, jnp.full((pad,), N, jnp.int32)])
    goff = (jnp.arange(GALL, dtype=jnp.int32) * N)[:, None]
    srcg = (src_p[None, :] + goff).reshape(GALL, NW, ITERS, K)
    dstp3 = dst_p.reshape(NW, ITERS, K)

    z16 = jnp.zeros((NP, 16), jnp.float32)
    zrows = jnp.zeros((NP, DH), jnp.float32)
    wx = W[:DIN, :]
    wh = W[DIN:, :]
    bias2 = b.reshape(1, G4)

    degp = _deg_kernel()(dstp3, z16)
    xs, dinv_b, sn_b = _scale_call(degp, xf)

    parts8 = _edge_scatter_kernel(GALL)(
        xs.reshape(GALL * N, DIN), srcg, dstp3, zrows)
    u_all = _u_call(parts8, xf, dinv_b, sn_b, wx, bias2)

    h, c, hs = _step0_call(u_all)(u_all, dinv_b)
    outs = [h]
    scat2 = _edge_scatter_kernel(B)
    for t in range(1, T):
        parts2 = scat2(hs.reshape(B * N, DH), srcg, dstp3, zrows)
        h, c, hs = _step_call(t)(u_all, parts2, h, c, dinv_b, sn_b, wh)
        outs.append(h)
    return jnp.stack(outs, axis=1)
